# Initial kernel scaffold; baseline (speedup 1.0000x reference)
import jax
import jax.numpy as jnp
from jax.experimental import pallas as pl


def kernel(probs, still_prompt, is_first, cur_pos, n_token_consider, n_token_sample, alive_seq, alive_log_probs, fin_seq, fin_log_probs):
    raise NotImplementedError("write your pallas kernel here")


# speedup vs baseline: 7.7139x; 7.7139x over previous
import functools

import jax
import jax.numpy as jnp
from jax import lax
from jax.experimental import pallas as pl
from jax.experimental.pallas import tpu as pltpu
from jax.experimental.pallas import tpu_sc as plsc

_INF = 1.0e7
_EOS_ID = 2
_LANES = 16
_K = 16
_CHUNK = 20000
_GROUP = 10


def _make_sc_topk(n_rows, vocab, n_drafts):
    n_prompts = n_rows // n_drafts
    n_chunks = vocab // _CHUNK
    assert n_chunks * _CHUNK == vocab
    groups_per_chunk = _CHUNK // (_GROUP * _LANES)
    assert groups_per_chunk * _GROUP * _LANES == _CHUNK

    mesh = plsc.VectorSubcoreMesh(core_axis_name="c", subcore_axis_name="s")

    @functools.partial(
        pl.kernel,
        out_type=(
            jax.ShapeDtypeStruct((n_prompts, n_drafts, _K), jnp.float32),
            jax.ShapeDtypeStruct((n_prompts, n_drafts, _K), jnp.int32),
        ),
        mesh=mesh,
        compiler_params=pltpu.CompilerParams(needs_layout_passes=False),
        scratch_types=[
            pltpu.VMEM((_CHUNK,), jnp.float32),
            pltpu.VMEM((_K,), jnp.float32),
            pltpu.VMEM((_K,), jnp.int32),
        ],
    )
    def sc_topk(probs_hbm, out_val, out_idx, buf, ov, oi):
        wid = lax.axis_index("s") * 2 + lax.axis_index("c")
        iota = lax.iota(jnp.int32, _LANES)

        def any_lane(mask):
            cnt = plsc.all_reduce_population_count(mask)
            return lax.squeeze(lax.slice(cnt, (0,), (1,)), (0,)) > 0

        def row_body(d, _):
            row_base = (wid * n_drafts + d) * vocab

            def chunk_body(c, carry):
                pltpu.sync_copy(probs_hbm.at[pl.ds(row_base + c * _CHUNK, _CHUNK)], buf)

                def group_body(g, carry):
                    r_val, r_idx, thresh = carry
                    gb = g * (_GROUP * _LANES)
                    vecs = [buf[pl.ds(gb + k * _LANES, _LANES)] for k in range(_GROUP)]
                    m = vecs[0]
                    for k in range(1, _GROUP):
                        m = jnp.maximum(m, vecs[k])

                    def rescan(carry):
                        for k in range(_GROUP):
                            v = vecs[k]

                            def merge(cc, v=v, k=k):
                                r_val, r_idx, _ = cc
                                vidx = (c * _CHUNK + gb + k * _LANES) + iota
                                sv, si = plsc.sort_key_val(v, vidx, descending=True)
                                take = sv > r_val
                                hi_v = jnp.where(take, sv, r_val)
                                hi_i = jnp.where(take, si, r_idx)
                                nv, ni = plsc.sort_key_val(hi_v, hi_i, descending=False)
                                new_t = lax.squeeze(lax.slice(nv, (0,), (1,)), (0,))
                                return (nv, ni, new_t)

                            carry = lax.cond(any_lane(v > carry[2]), merge,
                                             lambda cc: cc, carry)
                        return carry

                    return lax.cond(any_lane(m > thresh), rescan,
                                    lambda cc: cc, (r_val, r_idx, thresh))

                return lax.fori_loop(0, groups_per_chunk, group_body, carry)

            carry0 = (jnp.zeros((_K,), jnp.float32),
                      jnp.zeros((_K,), jnp.int32),
                      jnp.float32(0.0))
            r_val, r_idx, _ = lax.fori_loop(0, n_chunks, chunk_body, carry0)
            ov[...] = r_val
            oi[...] = r_idx
            pltpu.sync_copy(ov, out_val.at[wid, d])
            pltpu.sync_copy(oi, out_idx.at[wid, d])
            return 0

        lax.fori_loop(0, n_drafts, row_body, 0)

    return sc_topk


def kernel(probs, still_prompt, is_first, cur_pos, n_token_consider,
           n_token_sample, alive_seq, alive_log_probs, fin_seq, fin_log_probs):
    n_prompts, n_drafts = alive_log_probs.shape
    vocab = probs.shape[-1]

    sc_topk = _make_sc_topk(probs.shape[0], vocab, n_drafts)
    cand_val, cand_idx = sc_topk(probs.reshape(-1))

    order = jnp.argsort(cand_idx, axis=-1)
    cand_val = jnp.take_along_axis(cand_val, order, axis=-1)
    cand_idx = jnp.take_along_axis(cand_idx, order, axis=-1)

    scores = alive_log_probs[:, :, None] + jnp.log(cand_val)
    scores_flat = scores.reshape(n_prompts, n_drafts * _K)
    idx_flat = cand_idx.reshape(n_prompts, n_drafts * _K)

    topk_log_probs, pos = jax.lax.top_k(scores_flat, 2 * n_drafts)
    topk_beam_id = pos // _K
    topk_idx = jnp.take_along_axis(idx_flat, pos, axis=1)

    topk_finished = topk_idx == _EOS_ID
    alive_scores = topk_log_probs + jnp.where(topk_finished, -_INF, 0.0)
    _, alive_sel = jax.lax.top_k(alive_scores, n_drafts)
    ids = jnp.take_along_axis(topk_beam_id, alive_sel, axis=1)

    ids = jnp.where(is_first[:, None], jnp.zeros_like(ids), ids)
    ids = jnp.where(still_prompt[:, None],
                    jnp.broadcast_to(jnp.arange(n_drafts, dtype=ids.dtype),
                                     (n_prompts, n_drafts)),
                    ids)
    return ids
